# baseline (speedup 1.0000x reference)
import functools

import jax
import jax.numpy as jnp
from jax import lax
from jax.experimental import pallas as pl
from jax.experimental.pallas import tpu as pltpu
from jax.experimental.pallas import tpu_sc as plsc

_V = 1000000
_D = 64
_NB = 32
_LOGW = 15
_CAP = 8192
_PW = 1024
_NPIECE = (1 << _LOGW) // _PW
_NR = 4


@functools.lru_cache(maxsize=None)
def _build(B: int):
    mesh = plsc.VectorSubcoreMesh(core_axis_name="c", subcore_axis_name="s")

    @functools.partial(
        pl.kernel,
        mesh=mesh,
        out_type=jax.ShapeDtypeStruct((B + 16, 128), jnp.float32),
        scratch_types=[
            pltpu.VMEM((_CAP,), jnp.int32),
            pltpu.VMEM((_CAP,), jnp.int32),
            pltpu.VMEM((_CAP + 16,), jnp.int32),
            pltpu.VMEM((_CAP + 16,), jnp.int32),
            pltpu.VMEM((_D, _PW), jnp.float32),
            pltpu.VMEM((_NR, 16, 128), jnp.float32),
            pltpu.VMEM((_NB,), jnp.int32),
            pltpu.SemaphoreType.DMA,
        ]
        + [pltpu.SemaphoreType.DMA] * _NR,
        compiler_params=pltpu.CompilerParams(
            use_tc_tiling_on_sc=True, needs_layout_passes=False
        ),
    )
    def k(ridx_hbm, pos_hbm, ct_hbm, tabt_hbm, out_hbm,
          rbuf, pbuf, mcol, mpos, strip, ring, ct_v, ssem, *wsems):
        c = lax.axis_index("c")
        s = lax.axis_index("s")
        w = 16 * c + s
        pltpu.async_copy(ct_hbm, ct_v, ssem).wait()
        iota = lax.iota(jnp.int32, 16)
        ct_lo = ct_v[pl.ds(0, 16)]
        ct_hi = ct_v[pl.ds(16, 16)]
        sel = jnp.where(jnp.full((16,), c == 0), ct_lo, ct_hi)
        n = jnp.sum(jnp.where(iota == s, sel, 0))
        pltpu.sync_copy(ridx_hbm.at[w], rbuf)
        pltpu.sync_copy(pos_hbm.at[w], pbuf)
        n16 = lax.div(n + 15, 16)

        def piece_body(p, G):
            p_lo = w * (1 << _LOGW) + p * _PW
            for g in range(8):
                pltpu.async_copy(
                    tabt_hbm.at[pl.ds(g * 8, 8), pl.ds(p_lo, _PW)],
                    strip.at[pl.ds(8 * g, 8)],
                    ssem,
                )

            def scan_body(i, cur):
                rv = rbuf[pl.ds(i * 16, 16)]
                pv = pbuf[pl.ds(i * 16, 16)]
                m = jnp.logical_and(rv >= p_lo, rv < p_lo + _PW)
                plsc.store_compressed(mcol.at[pl.ds(cur, 16)], rv - p_lo, mask=m)
                plsc.store_compressed(mpos.at[pl.ds(cur, 16)], pv, mask=m)
                return cur + jnp.sum(m.astype(jnp.int32))

            cur = lax.fori_loop(0, n16, scan_body, 0)
            pltpu.make_async_copy(
                tabt_hbm.at[pl.ds(0, _D), pl.ds(0, _PW)], strip, ssem
            ).wait()

            def gather_body(j, gg):
                colv = mcol[pl.ds(j * 16, 16)]
                posv = mpos[pl.ds(j * 16, 16)]
                valid = iota + j * 16 < cur
                colv = jnp.where(valid, colv, 0)
                posv = jnp.where(valid, posv, B)
                b = lax.rem(gg, _NR)
                for rb in range(_NR):

                    @pl.when(jnp.logical_and(b == rb, gg >= _NR))
                    def _():
                        pltpu.make_async_copy(
                            out_hbm.at[pl.ds(0, 16)], ring.at[rb], wsems[rb]
                        ).wait()

                bvec = jnp.full((16,), b, jnp.int32)
                for dd in range(_D):
                    vals = plsc.load_gather(
                        strip, [jnp.full((16,), dd, jnp.int32), colv]
                    )
                    plsc.store_scatter(
                        ring, [bvec, iota, jnp.full((16,), dd, jnp.int32)], vals
                    )
                for rb in range(_NR):

                    @pl.when(b == rb)
                    def _():
                        pltpu.async_copy(
                            ring.at[rb], out_hbm.at[posv], wsems[rb]
                        )
                return gg + 1

            return lax.fori_loop(0, lax.div(cur + 15, 16), gather_body, G)

        gtot = lax.fori_loop(0, _NPIECE, piece_body, 0)
        pend = jnp.minimum(gtot, _NR)
        for rb in range(_NR):

            @pl.when(rb < pend)
            def _():
                pltpu.make_async_copy(
                    out_hbm.at[pl.ds(0, 16)], ring.at[rb], wsems[rb]
                ).wait()

    return k


def kernel(inputs, embeddings):
    rows, cols = inputs.shape
    B = rows * cols
    flat = inputs.reshape(B).astype(jnp.int32)
    owner = lax.shift_right_logical(flat, _LOGW)
    oh = (owner[:, None] == jnp.arange(_NB, dtype=jnp.int32)[None, :])
    rank = (
        jnp.take_along_axis(jnp.cumsum(oh.astype(jnp.int32), axis=0),
                            owner[:, None], axis=1)[:, 0] - 1
    )
    dest = owner * _CAP + jnp.minimum(rank, _CAP - 1)
    base_r = jnp.broadcast_to(
        (jnp.arange(_NB, dtype=jnp.int32) << _LOGW)[:, None], (_NB, _CAP)
    ).reshape(-1)
    init = jnp.stack([base_r, jnp.full((_NB * _CAP,), B, jnp.int32)], axis=1)
    pairs = jnp.stack([flat, jnp.arange(B, dtype=jnp.int32)], axis=1)
    part = init.at[dest].set(pairs, unique_indices=True)
    ridx = part[:, 0].reshape(_NB, _CAP)
    pos = part[:, 1].reshape(_NB, _CAP)
    counts = jnp.minimum(jnp.sum(oh, axis=0).astype(jnp.int32), _CAP)
    out = _build(B)(ridx, pos, counts, embeddings.T)
    return out[:B, :_D].reshape(rows, cols, _D)


# speedup vs baseline: 3.0197x; 3.0197x over previous
import functools

import jax
import jax.numpy as jnp
from jax import lax
from jax.experimental import pallas as pl
from jax.experimental.pallas import tpu as pltpu
from jax.experimental.pallas import tpu_sc as plsc

_DIM = 64
_CHUNK = 128
_NC = 2
_NS = 16
_NW = _NC * _NS


_NBUF = 8


@functools.lru_cache(maxsize=None)
def _build_gather(B: int):
    assert B % (_NW * _CHUNK) == 0
    b_per_w = B // _NW
    n_chunks = b_per_w // _CHUNK
    mesh = plsc.VectorSubcoreMesh(core_axis_name="c", subcore_axis_name="s")

    @functools.partial(
        pl.kernel,
        mesh=mesh,
        out_type=jax.ShapeDtypeStruct((B, _DIM), jnp.float32),
        scratch_types=[
            pltpu.VMEM((n_chunks, _CHUNK), jnp.int32),
            pltpu.VMEM((_NBUF, _CHUNK, _DIM), jnp.float32),
        ]
        + [pltpu.SemaphoreType.DMA] * (2 * _NBUF),
        compiler_params=pltpu.CompilerParams(use_tc_tiling_on_sc=False),
    )
    def gather_kernel(idx_hbm, table_hbm, out_hbm, idx_v, rows_v, *sems):
        gsem = sems[:_NBUF]
        wsem = sems[_NBUF:]
        wid = lax.axis_index("s") * _NC + lax.axis_index("c")
        base = wid * b_per_w
        pltpu.sync_copy(idx_hbm.at[wid], idx_v)

        def gather(ch, b):
            for j in range(_CHUNK // 16):
                vec = idx_v[ch, pl.ds(16 * j, 16)]
                pltpu.async_copy(
                    table_hbm.at[vec], rows_v.at[b, pl.ds(16 * j, 16)], gsem[b]
                )

        def write(ch, b):
            return pltpu.async_copy(
                rows_v.at[b], out_hbm.at[pl.ds(base + ch * _CHUNK, _CHUNK)],
                wsem[b],
            )

        for ch in range(_NBUF):
            gather(ch, ch)
        for ch in range(n_chunks):
            b = ch % _NBUF
            pltpu.make_async_copy(
                table_hbm.at[idx_v.at[ch]], rows_v.at[b], gsem[b]
            ).wait()
            w = write(ch, b)
            nxt = ch + _NBUF
            if nxt < n_chunks:
                w.wait()
                gather(nxt, b)
        for ch in range(n_chunks - _NBUF, n_chunks):
            b = ch % _NBUF
            pltpu.make_async_copy(
                rows_v.at[b], out_hbm.at[pl.ds(base + ch * _CHUNK, _CHUNK)],
                wsem[b],
            ).wait()

    return gather_kernel


def kernel(inputs, embeddings):
    rows, cols = inputs.shape
    B = rows * cols
    idx = inputs.reshape(_NW, B // (_NW * _CHUNK), _CHUNK).astype(jnp.int32)
    out = _build_gather(B)(idx, embeddings)
    return out.reshape(rows, cols, _DIM)
